# baseline (speedup 1.0000x reference)
import jax
import jax.numpy as jnp
from jax.experimental import pallas as pl
from jax.experimental.pallas import tpu as pltpu

_B, _F, _P, _V = 16, 32, 2, 6890
_L = _V * 3
_RB = _F


_CHUNK_V = 128
_CHUNK_L = 3 * _CHUNK_V
_NFULL = _V // _CHUNK_V
_REM_V = _V - _NFULL * _CHUNK_V
_REM_L = 3 * _REM_V


def _contact_body(a_ref, b_ref, cam_ref, gc_ref, sum_ref, cnt_ref):
    a = a_ref[:, 0, 0, :]
    b = b_ref[:, 0, 0, :]
    c = cam_ref[...]
    cd = c[:, 0:3] - c[:, 3:6]
    lane = jax.lax.broadcasted_iota(jnp.int32, (_RB, _L), 1)
    r = lane % 3
    cf = jnp.where(r == 0, cd[:, 0:1],
                   jnp.where(r == 1, cd[:, 1:2], cd[:, 2:3]))
    d = a - b + cf
    d2 = d * d
    m = (gc_ref[:, 0, 0, :] > 0).astype(jnp.float32)

    ri = jax.lax.broadcasted_iota(jnp.int32, (_CHUNK_L, _CHUNK_V), 0)
    ci = jax.lax.broadcasted_iota(jnp.int32, (_CHUNK_L, _CHUNK_V), 1)
    sel = (ri // 3 == ci).astype(jnp.float32)

    dn = (((1,), (0,)), ((), ()))
    acc = jnp.zeros((_RB, _CHUNK_V), jnp.float32)
    for j in range(_NFULL):
        d2c = d2[:, j * _CHUNK_L:(j + 1) * _CHUNK_L]
        mc = m[:, j * _CHUNK_V:(j + 1) * _CHUNK_V]
        sv = jax.lax.dot_general(d2c, sel, dn,
                                 preferred_element_type=jnp.float32)
        acc = acc + jnp.sqrt(sv) * mc
    d2r = d2[:, _NFULL * _CHUNK_L:]
    svr = jax.lax.dot_general(d2r, sel[:_REM_L, :_REM_V], dn,
                              preferred_element_type=jnp.float32)
    mr = m[:, _NFULL * _CHUNK_V:]
    s = jnp.sum(acc) + jnp.sum(jnp.sqrt(svr) * mr)
    n = jnp.sum(m)
    sum_ref[...] = jnp.broadcast_to(s, (1, 8, 128))
    cnt_ref[...] = jnp.broadcast_to(n, (1, 8, 128))


def kernel(pred_verts, pert_cam_t, dshape, gt_contact, valid):
    pv2 = pred_verts.reshape(_B * _F, _P, 1, _L)
    cam2 = pert_cam_t.reshape(_B * _F, _P * 3)
    gc2 = gt_contact.astype(jnp.int32).reshape(_B * _F, _P, 1, _V)

    out_shape = [
        jax.ShapeDtypeStruct((_B, 8, 128), jnp.float32),
        jax.ShapeDtypeStruct((_B, 8, 128), jnp.float32),
    ]
    sums, cnts = pl.pallas_call(
        _contact_body,
        grid=(_B,),
        in_specs=[
            pl.BlockSpec((_RB, 1, 1, _L), lambda i: (i, 0, 0, 0)),
            pl.BlockSpec((_RB, 1, 1, _L), lambda i: (i, 1, 0, 0)),
            pl.BlockSpec((_RB, 6), lambda i: (i, 0)),
            pl.BlockSpec((_RB, 1, 1, _V), lambda i: (i, 0, 0, 0)),
        ],
        out_specs=[
            pl.BlockSpec((1, 8, 128), lambda i: (i, 0, 0)),
            pl.BlockSpec((1, 8, 128), lambda i: (i, 0, 0)),
        ],
        out_shape=out_shape,
        compiler_params=pltpu.CompilerParams(
            dimension_semantics=("arbitrary",),
        ),
    )(pv2, pv2, cam2, gc2)

    s = sums[:, 0, 0]
    n = cnts[:, 0, 0]
    per_b_mean = jnp.where(n > 0, s / jnp.maximum(n, 1.0), 0.0)
    loss = jnp.sum(per_b_mean) / dshape[0].astype(jnp.float32)
    return loss * 10.0


# speedup vs baseline: 466.9669x; 466.9669x over previous
import jax
import jax.numpy as jnp
from jax.experimental import pallas as pl
from jax.experimental.pallas import tpu as pltpu

_B, _F, _P, _V = 16, 32, 2, 6890
_R = _B * _F * _P
_VB = 512
_NSTEP = (_V + _VB - 1) // _VB


def _contact_body(pv_ref, cam_ref, gc_ref, sum_ref, cnt_ref):
    i = pl.program_id(0)

    @pl.when(i == 0)
    def _init():
        sum_ref[...] = jnp.zeros_like(sum_ref)
        cnt_ref[...] = jnp.zeros_like(cnt_ref)

    t = pv_ref[...] + cam_ref[...][:, None, :]
    d = t - pltpu.roll(t, shift=_R - 1, axis=2)
    d2 = d * d
    sv = d2[0] + d2[1] + d2[2]
    dist = jnp.sqrt(sv)

    lane = jax.lax.broadcasted_iota(jnp.int32, (_VB, _R), 1)
    vtx = jax.lax.broadcasted_iota(jnp.int32, (_VB, _R), 0) + i * _VB
    valid = (lane % 2 == 0) & (vtx < _V) & (gc_ref[...] > 0)
    contrib = jnp.where(valid, dist, 0.0)
    ones = jnp.where(valid, 1.0, 0.0)

    psum = jnp.sum(contrib, axis=0, keepdims=True)
    pcnt = jnp.sum(ones, axis=0, keepdims=True)
    sum_ref[...] += jnp.broadcast_to(psum, (8, _R))
    cnt_ref[...] += jnp.broadcast_to(pcnt, (8, _R))


def kernel(pred_verts, pert_cam_t, dshape, gt_contact, valid):
    pvT = jnp.transpose(pred_verts, (2, 1, 0))
    camT = jnp.transpose(pert_cam_t, (1, 0))
    gcT = jnp.transpose(gt_contact.astype(jnp.int32), (1, 0))

    sums, cnts = pl.pallas_call(
        _contact_body,
        grid=(_NSTEP,),
        in_specs=[
            pl.BlockSpec((3, _VB, _R), lambda i: (0, i, 0)),
            pl.BlockSpec((3, _R), lambda i: (0, 0)),
            pl.BlockSpec((_VB, _R), lambda i: (i, 0)),
        ],
        out_specs=[
            pl.BlockSpec((8, _R), lambda i: (0, 0)),
            pl.BlockSpec((8, _R), lambda i: (0, 0)),
        ],
        out_shape=[
            jax.ShapeDtypeStruct((8, _R), jnp.float32),
            jax.ShapeDtypeStruct((8, _R), jnp.float32),
        ],
        compiler_params=pltpu.CompilerParams(
            dimension_semantics=("arbitrary",),
        ),
    )(pvT, camT, gcT)

    s_b = sums[0].reshape(_B, _F * _P).sum(axis=1)
    n_b = cnts[0].reshape(_B, _F * _P).sum(axis=1)
    per_b_mean = jnp.where(n_b > 0, s_b / jnp.maximum(n_b, 1.0), 0.0)
    loss = jnp.sum(per_b_mean) / dshape[0].astype(jnp.float32)
    return loss * 10.0


# speedup vs baseline: 484.8557x; 1.0383x over previous
import jax
import jax.numpy as jnp
from jax.experimental import pallas as pl
from jax.experimental.pallas import tpu as pltpu

_B, _F, _P, _V = 16, 32, 2, 6890
_R = _B * _F * _P
_VB = 696
_NSTEP = (_V + _VB - 1) // _VB


def _contact_body(pv_ref, cam_ref, gc_ref, sum_ref, cnt_ref):
    i = pl.program_id(0)

    @pl.when(i == 0)
    def _init():
        sum_ref[...] = jnp.zeros_like(sum_ref)
        cnt_ref[...] = jnp.zeros_like(cnt_ref)

    t = pv_ref[...] + cam_ref[...][:, None, :]
    d = t - pltpu.roll(t, shift=_R - 1, axis=2)
    d2 = d * d
    sv = d2[0] + d2[1] + d2[2]
    dist = jnp.sqrt(sv)

    lane = jax.lax.broadcasted_iota(jnp.int32, (_VB, _R), 1)
    vtx = jax.lax.broadcasted_iota(jnp.int32, (_VB, _R), 0) + i * _VB
    valid = (lane % 2 == 0) & (vtx < _V) & (gc_ref[...] > 0)
    contrib = jnp.where(valid, dist, 0.0)
    ones = jnp.where(valid, 1.0, 0.0)

    psum = jnp.sum(contrib, axis=0, keepdims=True)
    pcnt = jnp.sum(ones, axis=0, keepdims=True)
    sum_ref[...] += jnp.broadcast_to(psum, (8, _R))
    cnt_ref[...] += jnp.broadcast_to(pcnt, (8, _R))


def kernel(pred_verts, pert_cam_t, dshape, gt_contact, valid):
    pvT = jnp.transpose(pred_verts, (2, 1, 0))
    camT = jnp.transpose(pert_cam_t, (1, 0))
    gcT = jnp.transpose(gt_contact.astype(jnp.int32), (1, 0))

    sums, cnts = pl.pallas_call(
        _contact_body,
        grid=(_NSTEP,),
        in_specs=[
            pl.BlockSpec((3, _VB, _R), lambda i: (0, i, 0)),
            pl.BlockSpec((3, _R), lambda i: (0, 0)),
            pl.BlockSpec((_VB, _R), lambda i: (i, 0)),
        ],
        out_specs=[
            pl.BlockSpec((8, _R), lambda i: (0, 0)),
            pl.BlockSpec((8, _R), lambda i: (0, 0)),
        ],
        out_shape=[
            jax.ShapeDtypeStruct((8, _R), jnp.float32),
            jax.ShapeDtypeStruct((8, _R), jnp.float32),
        ],
        compiler_params=pltpu.CompilerParams(
            dimension_semantics=("arbitrary",),
        ),
    )(pvT, camT, gcT)

    s_b = sums[0].reshape(_B, _F * _P).sum(axis=1)
    n_b = cnts[0].reshape(_B, _F * _P).sum(axis=1)
    per_b_mean = jnp.where(n_b > 0, s_b / jnp.maximum(n_b, 1.0), 0.0)
    loss = jnp.sum(per_b_mean) / dshape[0].astype(jnp.float32)
    return loss * 10.0
